# baseline (speedup 1.0000x reference)
import functools

import jax
import jax.numpy as jnp
from jax import lax
from jax.experimental import pallas as pl
from jax.experimental.pallas import tpu as pltpu
from jax.experimental.pallas import tpu_sc as plsc

VOCAB = 1000000
EMBED = 64
BATCH = 4096
SEQ = 200

NUM_CORES = 2
NUM_SUBCORES = 16
NUM_WORKERS = NUM_CORES * NUM_SUBCORES
SB = SEQ // 8
BB = BATCH // 128
CHUNK = 256
N_STEPS = SB * 1024 // CHUNK
OPAD = CHUNK + 1
D_BLK = EMBED // 16


VPAD = 1000064
VB = VPAD // 128


@functools.partial(
    pl.kernel,
    out_type=jax.ShapeDtypeStruct((VB, 128, EMBED), jnp.float32),
    mesh=plsc.VectorSubcoreMesh(core_axis_name="c", subcore_axis_name="s"),
    compiler_params=pltpu.CompilerParams(
        needs_layout_passes=False, use_tc_tiling_on_sc=False
    ),
    scratch_types=[
        pltpu.VMEM((2, 8, 8, 128), jnp.float32),
        pltpu.VMEM((2, 128, 64), jnp.float32),
        pltpu.SemaphoreType.DMA,
        pltpu.SemaphoreType.DMA,
        pltpu.SemaphoreType.DMA,
        pltpu.SemaphoreType.DMA,
    ],
)
def _tok_transpose(p_hbm, v_hbm, t_v, o_v, si0, si1, so0, so1):
    wid = lax.axis_index("s") * NUM_CORES + lax.axis_index("c")
    si = (si0, si1)
    so = (so0, so1)
    n_iter = (VB + NUM_WORKERS - 1) // NUM_WORKERS

    lane = lax.iota(jnp.int32, 16)

    def issue_in(k, q):
        vb = NUM_WORKERS * k + wid
        pltpu.async_copy(p_hbm.at[:, vb], t_v.at[q], si[q])

    def wait_in(q):
        pltpu.make_async_copy(p_hbm.at[:, 0], t_v.at[q], si[q]).wait()

    def issue_out(k, q):
        vb = NUM_WORKERS * k + wid
        pltpu.async_copy(o_v.at[q], v_hbm.at[vb], so[q])

    def wait_out(q):
        pltpu.make_async_copy(o_v.at[q], v_hbm.at[0], so[q]).wait()

    def compute(q):
        ob = o_v.at[q]

        @plsc.parallel_loop(0, 8, unroll=1)
        def vl_body(c):
            rows = c * 16 + lane
            for dblk in range(8):
                for dr in range(8):
                    x = t_v[q, dblk, dr, pl.ds(c * 16, 16)]
                    col = jnp.full((16,), dblk * 8 + dr, jnp.int32)
                    plsc.store_scatter(ob, [rows, col], x)

    issue_in(0, 0)

    def loop_body(k2, carry):
        k = 2 * k2
        active0 = NUM_WORKERS * k + wid < VB
        active1 = NUM_WORKERS * (k + 1) + wid < VB

        @pl.when(active1)
        def _():
            issue_in(k + 1, 1)

        @pl.when(active0)
        def _():
            wait_in(0)

            @pl.when(k2 > 0)
            def _():
                wait_out(0)

            compute(0)
            issue_out(k, 0)

        @pl.when(NUM_WORKERS * (k + 2) + wid < VB)
        def _():
            issue_in(k + 2, 0)

        @pl.when(active1)
        def _():
            wait_in(1)

            @pl.when(k2 > 0)
            def _():
                wait_out(1)

            compute(1)
            issue_out(k + 1, 1)

        return carry

    lax.fori_loop(0, (n_iter + 1) // 2, loop_body, 0, unroll=False)
    wait_out(0)
    wait_out(1)


@functools.partial(
    pl.kernel,
    out_type=jax.ShapeDtypeStruct((EMBED, SB, BB, 1024), jnp.float32),
    mesh=plsc.VectorSubcoreMesh(core_axis_name="c", subcore_axis_name="s"),
    compiler_params=pltpu.CompilerParams(
        needs_layout_passes=False, use_tc_tiling_on_sc=False
    ),
    scratch_types=[
        pltpu.VMEM((SB, 1024), jnp.int32),
        pltpu.VMEM((SEQ, EMBED), jnp.float32),
        pltpu.VMEM((2, CHUNK, 2, 32), jnp.float32),
        pltpu.VMEM((2, EMBED, OPAD), jnp.float32),
        pltpu.SemaphoreType.DMA,
        pltpu.SemaphoreType.DMA,
        pltpu.SemaphoreType.DMA,
        pltpu.SemaphoreType.DMA,
    ],
)
def _tpe_sc(x_hbm, tok_hbm, pos_hbm, out_hbm, idx_v, pos_v, row_v, out_v,
            sg0, sg1, so0, so1):
    wid = lax.axis_index("s") * NUM_CORES + lax.axis_index("c")
    sg = (sg0, sg1)
    so = (so0, so1)

    pltpu.sync_copy(pos_hbm, pos_v)
    pltpu.sync_copy(x_hbm.at[:, wid], idx_v)

    lane = lax.iota(jnp.int32, 16)
    d_rows = [d0 * 16 + lane for d0 in range(D_BLK)]

    def issue_gathers(step, q):
        sb = step // 4
        qq = step % 4
        for h in range(2):
            pltpu.async_copy(
                tok_hbm.at[idx_v.at[sb, pl.ds(qq * CHUNK + h * 128, 128)]],
                row_v.at[q, pl.ds(h * 128, 128)], sg[q])

    def wait_gathers(q):
        for h in range(2):
            pltpu.make_async_copy(
                tok_hbm.at[pl.ds(0, 128)],
                row_v.at[q, pl.ds(h * 128, 128)], sg[q]
            ).wait()

    def issue_out(step, p):
        sb = step // 4
        qq = step % 4
        pltpu.async_copy(
            out_v.at[p].at[:, pl.ds(0, CHUNK)],
            out_hbm.at[:, sb, wid].at[:, pl.ds(qq * CHUNK, CHUNK)], so[p])

    def wait_out(p):
        pltpu.make_async_copy(
            out_v.at[p].at[:, pl.ds(0, CHUNK)],
            out_hbm.at[:, 0, wid].at[:, pl.ds(0, CHUNK)], so[p]
        ).wait()

    def compute_into(step, p, q):
        ob = out_v.at[p]
        sb = step // 4
        qq = step % 4
        for h in range(2):
            s = 8 * sb + 2 * qq + h
            pvec = [pos_v[s, pl.ds(d0 * 16, 16)] for d0 in range(D_BLK)]

            @plsc.parallel_loop(0, 128, unroll=4)
            def bl_body(bl):
                r = h * 128 + bl
                cvec = jnp.full((16,), 0, jnp.int32) + r
                for d0 in range(D_BLK):
                    v = (row_v[q, r, d0 // 2, pl.ds((d0 % 2) * 16, 16)]
                         + pvec[d0])
                    plsc.store_scatter(ob, [d_rows[d0], cvec], v)

    issue_gathers(0, 0)

    def pair_body(j2, carry):
        step = 2 * j2
        issue_gathers(step + 1, 1)
        wait_gathers(0)

        @pl.when(j2 > 0)
        def _():
            wait_out(0)

        compute_into(step, 0, 0)
        issue_out(step, 0)

        @pl.when(j2 < N_STEPS // 2 - 1)
        def _():
            issue_gathers(step + 2, 0)

        wait_gathers(1)

        @pl.when(j2 > 0)
        def _():
            wait_out(1)

        compute_into(step + 1, 1, 1)
        issue_out(step + 1, 1)
        return carry

    lax.fori_loop(0, N_STEPS // 2, pair_body, 0, unroll=False)
    wait_out(0)
    wait_out(1)


def kernel(x, token_table, pos_table):
    x4 = (x.astype(jnp.int32).T.reshape(SB, 8, BB, 128)
          .transpose(0, 2, 1, 3).reshape(SB, BB, 1024))
    tok_pad = jnp.pad(token_table, ((0, VPAD - VOCAB), (0, 0)))
    p4 = (tok_pad.T.reshape(8, 8, VB, 128)
          .transpose(0, 2, 1, 3))
    tok_vmajor = _tok_transpose(p4)
    tok3 = tok_vmajor.reshape(VPAD, 2, 32)
    o = _tpe_sc(x4, tok3, pos_table)
    return (o.reshape(EMBED, SB, BB, 8, 128)
            .transpose(2, 4, 0, 1, 3).reshape(BATCH, EMBED, SEQ))


# speedup vs baseline: 2.2431x; 2.2431x over previous
import functools

import jax
import jax.numpy as jnp
from jax import lax
from jax.experimental import pallas as pl
from jax.experimental.pallas import tpu as pltpu
from jax.experimental.pallas import tpu_sc as plsc

VOCAB = 1000000
EMBED = 64
BATCH = 4096
SEQ = 200

NUM_CORES = 2
NUM_SUBCORES = 16
NUM_WORKERS = NUM_CORES * NUM_SUBCORES
SB = SEQ // 8
BB = BATCH // 128
CHUNK = 256
N_STEPS = SB * 1024 // CHUNK
OPAD = CHUNK + 1
D_BLK = EMBED // 16


VPAD = 1000064
VB = VPAD // 128


@functools.partial(
    pl.kernel,
    out_type=jax.ShapeDtypeStruct((VB, 128, EMBED), jnp.float32),
    mesh=plsc.VectorSubcoreMesh(core_axis_name="c", subcore_axis_name="s"),
    compiler_params=pltpu.CompilerParams(
        needs_layout_passes=False, use_tc_tiling_on_sc=False
    ),
    scratch_types=[
        pltpu.VMEM((2, 8, 8, 128), jnp.float32),
        pltpu.VMEM((2, 128, 64), jnp.float32),
        pltpu.SemaphoreType.DMA,
        pltpu.SemaphoreType.DMA,
        pltpu.SemaphoreType.DMA,
        pltpu.SemaphoreType.DMA,
    ],
)
def _tok_transpose(p_hbm, v_hbm, t_v, o_v, si0, si1, so0, so1):
    wid = lax.axis_index("s") * NUM_CORES + lax.axis_index("c")
    si = (si0, si1)
    so = (so0, so1)
    n_iter = (VB + NUM_WORKERS - 1) // NUM_WORKERS

    lane = lax.iota(jnp.int32, 16)

    def issue_in(k, q):
        vb = NUM_WORKERS * k + wid
        pltpu.async_copy(p_hbm.at[:, vb], t_v.at[q], si[q])

    def wait_in(q):
        pltpu.make_async_copy(p_hbm.at[:, 0], t_v.at[q], si[q]).wait()

    def issue_out(k, q):
        vb = NUM_WORKERS * k + wid
        pltpu.async_copy(o_v.at[q], v_hbm.at[vb], so[q])

    def wait_out(q):
        pltpu.make_async_copy(o_v.at[q], v_hbm.at[0], so[q]).wait()

    def compute(q):
        ob = o_v.at[q]
        tb = t_v.at[q]

        @plsc.parallel_loop(0, 16, unroll=1)
        def j_body(j):
            perm = lane + j
            perm = jnp.where(perm >= 16, perm - 16, perm)
            ph = perm >> 3
            pm7 = perm & 7
            for c in range(8):
                vrow = c * 16 + lane
                for d0 in range(4):
                    x = plsc.load_gather(tb, [2 * d0 + ph, pm7, vrow])
                    plsc.store_scatter(ob, [vrow, d0 * 16 + perm], x)

    issue_in(0, 0)

    def loop_body(k2, carry):
        k = 2 * k2
        active0 = NUM_WORKERS * k + wid < VB
        active1 = NUM_WORKERS * (k + 1) + wid < VB

        @pl.when(active1)
        def _():
            issue_in(k + 1, 1)

        @pl.when(active0)
        def _():
            wait_in(0)

            @pl.when(k2 > 0)
            def _():
                wait_out(0)

            compute(0)
            issue_out(k, 0)

        @pl.when(NUM_WORKERS * (k + 2) + wid < VB)
        def _():
            issue_in(k + 2, 0)

        @pl.when(active1)
        def _():
            wait_in(1)

            @pl.when(k2 > 0)
            def _():
                wait_out(1)

            compute(1)
            issue_out(k + 1, 1)

        return carry

    lax.fori_loop(0, (n_iter + 1) // 2, loop_body, 0, unroll=False)
    wait_out(0)
    wait_out(1)


@functools.partial(
    pl.kernel,
    out_type=jax.ShapeDtypeStruct((EMBED, SB, BB, 1024), jnp.float32),
    mesh=plsc.VectorSubcoreMesh(core_axis_name="c", subcore_axis_name="s"),
    compiler_params=pltpu.CompilerParams(
        needs_layout_passes=False, use_tc_tiling_on_sc=False
    ),
    scratch_types=[
        pltpu.VMEM((SB, 1024), jnp.int32),
        pltpu.VMEM((SEQ, EMBED), jnp.float32),
        pltpu.VMEM((2, CHUNK, 2, 32), jnp.float32),
        pltpu.VMEM((2, EMBED, OPAD), jnp.float32),
        pltpu.SemaphoreType.DMA,
        pltpu.SemaphoreType.DMA,
        pltpu.SemaphoreType.DMA,
        pltpu.SemaphoreType.DMA,
    ],
)
def _tpe_sc(x_hbm, tok_hbm, pos_hbm, out_hbm, idx_v, pos_v, row_v, out_v,
            sg0, sg1, so0, so1):
    wid = lax.axis_index("s") * NUM_CORES + lax.axis_index("c")
    sg = (sg0, sg1)
    so = (so0, so1)

    pltpu.sync_copy(pos_hbm, pos_v)
    pltpu.sync_copy(x_hbm.at[:, wid], idx_v)

    lane = lax.iota(jnp.int32, 16)
    d_rows = [d0 * 16 + lane for d0 in range(D_BLK)]

    def issue_gathers(step, q):
        sb = step // 4
        qq = step % 4
        for h in range(2):
            pltpu.async_copy(
                tok_hbm.at[idx_v.at[sb, pl.ds(qq * CHUNK + h * 128, 128)]],
                row_v.at[q, pl.ds(h * 128, 128)], sg[q])

    def wait_gathers(q):
        for h in range(2):
            pltpu.make_async_copy(
                tok_hbm.at[pl.ds(0, 128)],
                row_v.at[q, pl.ds(h * 128, 128)], sg[q]
            ).wait()

    def issue_out(step, p):
        sb = step // 4
        qq = step % 4
        pltpu.async_copy(
            out_v.at[p].at[:, pl.ds(0, CHUNK)],
            out_hbm.at[:, sb, wid].at[:, pl.ds(qq * CHUNK, CHUNK)], so[p])

    def wait_out(p):
        pltpu.make_async_copy(
            out_v.at[p].at[:, pl.ds(0, CHUNK)],
            out_hbm.at[:, 0, wid].at[:, pl.ds(0, CHUNK)], so[p]
        ).wait()

    def compute_into(step, p, q):
        ob = out_v.at[p]
        sb = step // 4
        qq = step % 4
        for h in range(2):
            s = 8 * sb + 2 * qq + h
            pvec = [pos_v[s, pl.ds(d0 * 16, 16)] for d0 in range(D_BLK)]

            @plsc.parallel_loop(0, 128, unroll=4)
            def bl_body(bl):
                r = h * 128 + bl
                cvec = jnp.full((16,), 0, jnp.int32) + r
                for d0 in range(D_BLK):
                    v = (row_v[q, r, d0 // 2, pl.ds((d0 % 2) * 16, 16)]
                         + pvec[d0])
                    plsc.store_scatter(ob, [d_rows[d0], cvec], v)

    issue_gathers(0, 0)

    def pair_body(j2, carry):
        step = 2 * j2
        issue_gathers(step + 1, 1)
        wait_gathers(0)

        @pl.when(j2 > 0)
        def _():
            wait_out(0)

        compute_into(step, 0, 0)
        issue_out(step, 0)

        @pl.when(j2 < N_STEPS // 2 - 1)
        def _():
            issue_gathers(step + 2, 0)

        wait_gathers(1)

        @pl.when(j2 > 0)
        def _():
            wait_out(1)

        compute_into(step + 1, 1, 1)
        issue_out(step + 1, 1)
        return carry

    lax.fori_loop(0, N_STEPS // 2, pair_body, 0, unroll=False)
    wait_out(0)
    wait_out(1)


def kernel(x, token_table, pos_table):
    x4 = (x.astype(jnp.int32).T.reshape(SB, 8, BB, 128)
          .transpose(0, 2, 1, 3).reshape(SB, BB, 1024))
    tok_pad = jnp.pad(token_table, ((0, VPAD - VOCAB), (0, 0)))
    p4 = (tok_pad.T.reshape(8, 8, VB, 128)
          .transpose(0, 2, 1, 3))
    tok_vmajor = _tok_transpose(p4)
    tok3 = tok_vmajor.reshape(VPAD, 2, 32)
    o = _tpe_sc(x4, tok3, pos_table)
    return (o.reshape(EMBED, SB, BB, 8, 128)
            .transpose(2, 4, 0, 1, 3).reshape(BATCH, EMBED, SEQ))


# speedup vs baseline: 2.2906x; 1.0212x over previous
import functools

import jax
import jax.numpy as jnp
from jax import lax
from jax.experimental import pallas as pl
from jax.experimental.pallas import tpu as pltpu
from jax.experimental.pallas import tpu_sc as plsc

VOCAB = 1000000
EMBED = 64
BATCH = 4096
SEQ = 200

NUM_CORES = 2
NUM_SUBCORES = 16
NUM_WORKERS = NUM_CORES * NUM_SUBCORES
SB = SEQ // 8
BB = BATCH // 128
CHUNK = 256
N_STEPS = SB * 1024 // CHUNK
OPAD = CHUNK + 1
D_BLK = EMBED // 16


VPAD = 1000064
VB = VPAD // 128


@functools.partial(
    pl.kernel,
    out_type=jax.ShapeDtypeStruct((VB, 128, EMBED), jnp.float32),
    mesh=plsc.VectorSubcoreMesh(core_axis_name="c", subcore_axis_name="s"),
    compiler_params=pltpu.CompilerParams(
        needs_layout_passes=False, use_tc_tiling_on_sc=False
    ),
    scratch_types=[
        pltpu.VMEM((2, 8, 8, 128), jnp.float32),
        pltpu.VMEM((2, 128, 64), jnp.float32),
        pltpu.SemaphoreType.DMA,
        pltpu.SemaphoreType.DMA,
        pltpu.SemaphoreType.DMA,
        pltpu.SemaphoreType.DMA,
    ],
)
def _tok_transpose(p_hbm, v_hbm, t_v, o_v, si0, si1, so0, so1):
    wid = lax.axis_index("s") * NUM_CORES + lax.axis_index("c")
    si = (si0, si1)
    so = (so0, so1)
    n_iter = (VB + NUM_WORKERS - 1) // NUM_WORKERS

    lane = lax.iota(jnp.int32, 16)

    def issue_in(k, q):
        vb = NUM_WORKERS * k + wid
        pltpu.async_copy(p_hbm.at[:, vb], t_v.at[q], si[q])

    def wait_in(q):
        pltpu.make_async_copy(p_hbm.at[:, 0], t_v.at[q], si[q]).wait()

    def issue_out(k, q):
        vb = NUM_WORKERS * k + wid
        pltpu.async_copy(o_v.at[q], v_hbm.at[vb], so[q])

    def wait_out(q):
        pltpu.make_async_copy(o_v.at[q], v_hbm.at[0], so[q]).wait()

    def compute(q):
        ob = o_v.at[q]
        tb = t_v.at[q]

        @plsc.parallel_loop(0, 16, unroll=2)
        def j_body(j):
            perm = lane + j
            perm = jnp.where(perm >= 16, perm - 16, perm)
            ph = perm >> 3
            pm7 = perm & 7
            for c in range(8):
                vrow = c * 16 + lane
                for d0 in range(4):
                    x = plsc.load_gather(tb, [2 * d0 + ph, pm7, vrow])
                    plsc.store_scatter(ob, [vrow, d0 * 16 + perm], x)

    issue_in(0, 0)

    def loop_body(k2, carry):
        k = 2 * k2
        active0 = NUM_WORKERS * k + wid < VB
        active1 = NUM_WORKERS * (k + 1) + wid < VB

        @pl.when(active1)
        def _():
            issue_in(k + 1, 1)

        @pl.when(active0)
        def _():
            wait_in(0)

            @pl.when(k2 > 0)
            def _():
                wait_out(0)

            compute(0)
            issue_out(k, 0)

        @pl.when(NUM_WORKERS * (k + 2) + wid < VB)
        def _():
            issue_in(k + 2, 0)

        @pl.when(active1)
        def _():
            wait_in(1)

            @pl.when(k2 > 0)
            def _():
                wait_out(1)

            compute(1)
            issue_out(k + 1, 1)

        return carry

    lax.fori_loop(0, (n_iter + 1) // 2, loop_body, 0, unroll=False)
    wait_out(0)
    wait_out(1)


@functools.partial(
    pl.kernel,
    out_type=jax.ShapeDtypeStruct((EMBED, SB, BB, 1024), jnp.float32),
    mesh=plsc.VectorSubcoreMesh(core_axis_name="c", subcore_axis_name="s"),
    compiler_params=pltpu.CompilerParams(
        needs_layout_passes=False, use_tc_tiling_on_sc=False
    ),
    scratch_types=[
        pltpu.VMEM((SB, 1024), jnp.int32),
        pltpu.VMEM((SEQ, EMBED), jnp.float32),
        pltpu.VMEM((2, CHUNK, 2, 32), jnp.float32),
        pltpu.VMEM((2, EMBED, OPAD), jnp.float32),
        pltpu.SemaphoreType.DMA,
        pltpu.SemaphoreType.DMA,
        pltpu.SemaphoreType.DMA,
        pltpu.SemaphoreType.DMA,
    ],
)
def _tpe_sc(x_hbm, tok_hbm, pos_hbm, out_hbm, idx_v, pos_v, row_v, out_v,
            sg0, sg1, so0, so1):
    wid = lax.axis_index("s") * NUM_CORES + lax.axis_index("c")
    sg = (sg0, sg1)
    so = (so0, so1)

    pltpu.sync_copy(pos_hbm, pos_v)
    pltpu.sync_copy(x_hbm.at[:, wid], idx_v)

    lane = lax.iota(jnp.int32, 16)
    d_rows = [d0 * 16 + lane for d0 in range(D_BLK)]

    def issue_gathers(step, q):
        sb = step // 4
        qq = step % 4
        for h in range(2):
            pltpu.async_copy(
                tok_hbm.at[idx_v.at[sb, pl.ds(qq * CHUNK + h * 128, 128)]],
                row_v.at[q, pl.ds(h * 128, 128)], sg[q])

    def wait_gathers(q):
        for h in range(2):
            pltpu.make_async_copy(
                tok_hbm.at[pl.ds(0, 128)],
                row_v.at[q, pl.ds(h * 128, 128)], sg[q]
            ).wait()

    def issue_out(step, p):
        sb = step // 4
        qq = step % 4
        pltpu.async_copy(
            out_v.at[p].at[:, pl.ds(0, CHUNK)],
            out_hbm.at[:, sb, wid].at[:, pl.ds(qq * CHUNK, CHUNK)], so[p])

    def wait_out(p):
        pltpu.make_async_copy(
            out_v.at[p].at[:, pl.ds(0, CHUNK)],
            out_hbm.at[:, 0, wid].at[:, pl.ds(0, CHUNK)], so[p]
        ).wait()

    def compute_into(step, p, q):
        ob = out_v.at[p]
        sb = step // 4
        qq = step % 4
        for h in range(2):
            s = 8 * sb + 2 * qq + h
            pvec = [pos_v[s, pl.ds(d0 * 16, 16)] for d0 in range(D_BLK)]

            @plsc.parallel_loop(0, 128, unroll=4)
            def bl_body(bl):
                r = h * 128 + bl
                cvec = jnp.full((16,), 0, jnp.int32) + r
                for d0 in range(D_BLK):
                    v = (row_v[q, r, d0 // 2, pl.ds((d0 % 2) * 16, 16)]
                         + pvec[d0])
                    plsc.store_scatter(ob, [d_rows[d0], cvec], v)

    issue_gathers(0, 0)

    def pair_body(j2, carry):
        step = 2 * j2
        issue_gathers(step + 1, 1)
        wait_gathers(0)

        @pl.when(j2 > 0)
        def _():
            wait_out(0)

        compute_into(step, 0, 0)
        issue_out(step, 0)

        @pl.when(j2 < N_STEPS // 2 - 1)
        def _():
            issue_gathers(step + 2, 0)

        wait_gathers(1)

        @pl.when(j2 > 0)
        def _():
            wait_out(1)

        compute_into(step + 1, 1, 1)
        issue_out(step + 1, 1)
        return carry

    lax.fori_loop(0, N_STEPS // 2, pair_body, 0, unroll=False)
    wait_out(0)
    wait_out(1)


def kernel(x, token_table, pos_table):
    x4 = (x.astype(jnp.int32).T.reshape(SB, 8, BB, 128)
          .transpose(0, 2, 1, 3).reshape(SB, BB, 1024))
    tok_pad = jnp.pad(token_table, ((0, VPAD - VOCAB), (0, 0)))
    p4 = (tok_pad.T.reshape(8, 8, VB, 128)
          .transpose(0, 2, 1, 3))
    tok_vmajor = _tok_transpose(p4)
    tok3 = tok_vmajor.reshape(VPAD, 2, 32)
    o = _tpe_sc(x4, tok3, pos_table)
    return (o.reshape(EMBED, SB, BB, 8, 128)
            .transpose(2, 4, 0, 1, 3).reshape(BATCH, EMBED, SEQ))
